# baseline (speedup 1.0000x reference)
import functools

import jax
import jax.numpy as jnp
from jax import lax
from jax.experimental import pallas as pl
from jax.experimental.pallas import tpu as pltpu
from jax.experimental.pallas import tpu_sc as plsc

LANES = 128


def _build_transpose(vocab, dim0, num_cores, num_subcores):
    nw = num_cores * num_subcores
    ch = 1024
    jw = ch // LANES
    vmain = (vocab // ch) * ch
    n_chunk = vmain // ch
    rounds = -(-n_chunk // nw)
    jpad = (n_chunk + 1) * jw
    vtail = vocab - vmain
    hrows = jw * (dim0 // 2)
    grows = jpad * (dim0 // 2)
    orows = ch * dim0 // LANES
    mesh = plsc.VectorSubcoreMesh(core_axis_name="c", subcore_axis_name="s")

    @functools.partial(
        pl.kernel,
        mesh=mesh,
        compiler_params=pltpu.CompilerParams(
            use_tc_tiling_on_sc=False, needs_layout_passes=False
        ),
        out_type=jax.ShapeDtypeStruct((vocab * dim0 // LANES, LANES), jnp.float32),
        scratch_types=[
            pltpu.VMEM((2, 2 * hrows, LANES), jnp.float32),
            pltpu.VMEM((2, orows, LANES), jnp.float32),
            pltpu.SemaphoreType.DMA,
            pltpu.SemaphoreType.DMA,
            pltpu.SemaphoreType.DMA,
            pltpu.SemaphoreType.DMA,
        ],
    )
    def tkernel(tlin_hbm, out_hbm, in_v, out_v, gs0, gs1, os0, os1):
        wid = lax.axis_index("s") * num_cores + lax.axis_index("c")
        iota16 = lax.iota(jnp.int32, dim0)
        rbase16 = (iota16 // 8) * hrows + (iota16 % 8)
        gsems = [gs0, gs1]
        osems = [os0, os1]

        def in_copies(r, buf):
            off = (r * nw + wid) * hrows
            return [
                pltpu.make_async_copy(
                    tlin_hbm.at[pl.ds(off, hrows)],
                    in_v.at[buf, pl.ds(0, hrows)],
                    gsems[buf],
                ),
                pltpu.make_async_copy(
                    tlin_hbm.at[pl.ds(grows + off, hrows)],
                    in_v.at[buf, pl.ds(hrows, hrows)],
                    gsems[buf],
                ),
            ]

        def out_copy(r, buf):
            return pltpu.make_async_copy(
                out_v.at[buf],
                out_hbm.at[pl.ds((r * nw + wid) * orows, orows)],
                osems[buf],
            )

        def transpose_chunk(buf, n_cols):
            def col_loop(jp, nc):
                rows_j = rbase16 + jp * jw

                def cbody(c, lanes):
                    col = plsc.load_gather(in_v.at[buf], [rows_j, lanes])
                    out_v[
                        buf,
                        jp * (LANES // 8) + c // 8,
                        pl.ds((c % 8) * dim0, dim0),
                    ] = col
                    return lanes + 1

                lax.fori_loop(0, nc, cbody, iota16 * 0, unroll=8)

            def jbody(jp, carry):
                col_loop(jp, LANES)
                return carry

            lax.fori_loop(0, n_cols // LANES, jbody, 0)
            if n_cols % LANES:
                col_loop(n_cols // LANES, n_cols % LANES)

        for cp in in_copies(0, 0):
            cp.start()

        for r in range(rounds):
            buf = r % 2
            nb = (r + 1) % 2

            @pl.when((r + 1) * nw + wid < n_chunk)
            def _():
                for cp in in_copies(r + 1, nb):
                    cp.start()

            if r >= 2:
                @pl.when((r - 2) * nw + wid < n_chunk)
                def _():
                    out_copy(r - 2, buf).wait()

            @pl.when(r * nw + wid < n_chunk)
            def _():
                for cp in in_copies(r, buf):
                    cp.wait()
                transpose_chunk(buf, ch)
                out_copy(r, buf).start()

        for r in (rounds - 2, rounds - 1):

            @pl.when(r * nw + wid < n_chunk)
            def _():
                out_copy(r, r % 2).wait()

        @pl.when(wid == nw - 1)
        def _():
            pltpu.sync_copy(
                tlin_hbm.at[pl.ds(n_chunk * hrows, hrows)],
                in_v.at[0, pl.ds(0, hrows)],
            )
            pltpu.sync_copy(
                tlin_hbm.at[pl.ds(grows + n_chunk * hrows, hrows)],
                in_v.at[0, pl.ds(hrows, hrows)],
            )
            transpose_chunk(0, vtail)
            pltpu.sync_copy(
                out_v.at[0, pl.ds(0, vtail * dim0 // LANES)],
                out_hbm.at[
                    pl.ds(vmain * dim0 // LANES, vtail * dim0 // LANES)
                ],
            )

    return tkernel, jpad


def _build_gather(n_idx, dim0, num_cores, num_subcores):
    n_workers = num_cores * num_subcores
    per_w = n_idx // n_workers
    chunk = 2048
    while per_w % chunk != 0:
        chunk //= 2
    n_chunk = per_w // chunk

    mesh = plsc.VectorSubcoreMesh(core_axis_name="c", subcore_axis_name="s")

    @functools.partial(
        pl.kernel,
        mesh=mesh,
        compiler_params=pltpu.CompilerParams(use_tc_tiling_on_sc=False, needs_layout_passes=False),
        out_type=jax.ShapeDtypeStruct((n_idx, dim0), jnp.float32),
        scratch_types=[
            pltpu.VMEM((per_w,), jnp.int32),
            pltpu.VMEM((2, chunk, dim0), jnp.float32),
            pltpu.SemaphoreType.DMA,
            pltpu.SemaphoreType.DMA,
        ],
    )
    def gather(idx_hbm, table_hbm, emb_hbm, idx_v, rows_v, gsem0, gsem1):
        wid = lax.axis_index("s") * num_cores + lax.axis_index("c")
        base = wid * per_w
        pltpu.sync_copy(idx_hbm.at[pl.ds(base, per_w)], idx_v)
        gsems = [gsem0, gsem1]
        copies = [None, None]
        copies[0] = pltpu.async_copy(
            table_hbm.at[idx_v.at[pl.ds(0, chunk)]], rows_v.at[0], gsems[0]
        )
        for j in range(n_chunk):
            bj = j % 2
            if j + 1 < n_chunk:
                bn = (j + 1) % 2
                copies[bn] = pltpu.async_copy(
                    table_hbm.at[idx_v.at[pl.ds((j + 1) * chunk, chunk)]],
                    rows_v.at[bn],
                    gsems[bn],
                )
            copies[bj].wait()
            pltpu.sync_copy(
                rows_v.at[bj], emb_hbm.at[pl.ds(base + j * chunk, chunk)]
            )

    return gather


def _make_mm_body(dim0, dim1, pack, q_blk):
    def _mm_body(emb_ref, w_ref, b_ref, out_ref):
        for c2 in range(pack):
            sub = emb_ref[:, c2 * dim0 : (c2 + 1) * dim0]
            acc = lax.dot_general(
                w_ref[...],
                sub,
                (((0,), (1,)), ((), ())),
                preferred_element_type=jnp.float32,
            )
            out_ref[0, :, pl.ds(c2 * q_blk, q_blk)] = acc + b_ref[...]

    return _mm_body


def _upscale_t(emb128, w, b_col, seq, bsz, b_blk):
    dim0, dim1 = w.shape
    pack = 128 // dim0
    q_blk = b_blk // pack
    nb = bsz // b_blk
    return pl.pallas_call(
        _make_mm_body(dim0, dim1, pack, q_blk),
        grid=(seq, nb),
        in_specs=[
            pl.BlockSpec((q_blk, 128), lambda l, j: (l * nb + j, 0)),
            pl.BlockSpec((dim0, dim1), lambda l, j: (0, 0)),
            pl.BlockSpec((dim1, 1), lambda l, j: (0, 0)),
        ],
        out_specs=pl.BlockSpec((1, dim1, b_blk), lambda l, j: (l, 0, j)),
        out_shape=jax.ShapeDtypeStruct((seq, dim1, bsz), jnp.float32),
    )(emb128, w, b_col)


def kernel(x, table, W, b):
    bsz, seq = x.shape
    vocab, dim0 = table.shape
    dim1 = W.shape[1]
    n_idx = bsz * seq

    info = plsc.get_sparse_core_info()
    nc, ns = info.num_cores, info.num_subcores

    tkernel, jpad = _build_transpose(vocab, dim0, nc, ns)
    ttp = jnp.pad(table.T, ((0, 0), (0, jpad * LANES - vocab)))
    tlin = (
        ttp.reshape(2, dim0 // 2, jpad, LANES)
        .transpose(0, 2, 1, 3)
        .reshape(2 * jpad * dim0 // 2, LANES)
    )
    table_lin = tkernel(tlin).reshape(vocab, dim0)

    idx = x.T.reshape(-1).astype(jnp.int32)
    b_blk = 16384
    pack = 128 // dim0
    q_blk = b_blk // pack
    n_blocks = n_idx // b_blk
    idx_p = idx.reshape(n_blocks, pack, q_blk).swapaxes(1, 2).reshape(-1)
    gather = _build_gather(n_idx, dim0, nc, ns)
    emb = gather(idx_p, table_lin)
    emb128 = emb.reshape(n_idx // pack, pack * dim0)

    out3 = _upscale_t(emb128, W, b.reshape(dim1, 1), seq, bsz, b_blk=b_blk)
    return out3.transpose(2, 0, 1)


# speedup vs baseline: 1.2555x; 1.2555x over previous
import functools

import jax
import jax.numpy as jnp
from jax import lax
from jax.experimental import pallas as pl
from jax.experimental.pallas import tpu as pltpu
from jax.experimental.pallas import tpu_sc as plsc

LANES = 128


def _build_transpose(vocab, dim0, num_cores, num_subcores):
    nw = num_cores * num_subcores
    ch = 1024
    jw = ch // LANES
    vmain = (vocab // ch) * ch
    n_chunk = vmain // ch
    rounds = -(-n_chunk // nw)
    jpad = (n_chunk + 1) * jw
    vtail = vocab - vmain
    hrows = jw * (dim0 // 2)
    grows = jpad * (dim0 // 2)
    orows = ch * dim0 // LANES
    mesh = plsc.VectorSubcoreMesh(core_axis_name="c", subcore_axis_name="s")

    @functools.partial(
        pl.kernel,
        mesh=mesh,
        compiler_params=pltpu.CompilerParams(
            use_tc_tiling_on_sc=False, needs_layout_passes=False
        ),
        out_type=jax.ShapeDtypeStruct((vocab, dim0), jnp.float32),
        scratch_types=[
            pltpu.VMEM((2, 2 * hrows, LANES), jnp.float32),
            pltpu.VMEM((2, ch, dim0 + 1), jnp.float32),
            pltpu.SemaphoreType.DMA,
            pltpu.SemaphoreType.DMA,
            pltpu.SemaphoreType.DMA,
            pltpu.SemaphoreType.DMA,
        ],
    )
    def tkernel(tlin_hbm, out_hbm, in_v, out_v, gs0, gs1, os0, os1):
        wid = lax.axis_index("s") * num_cores + lax.axis_index("c")
        iota16 = lax.iota(jnp.int32, dim0)
        rbase16 = (iota16 // 8) * hrows + (iota16 % 8)
        gsems = [gs0, gs1]
        osems = [os0, os1]

        def in_copies(r, buf):
            off = (r * nw + wid) * hrows
            return [
                pltpu.make_async_copy(
                    tlin_hbm.at[pl.ds(off, hrows)],
                    in_v.at[buf, pl.ds(0, hrows)],
                    gsems[buf],
                ),
                pltpu.make_async_copy(
                    tlin_hbm.at[pl.ds(grows + off, hrows)],
                    in_v.at[buf, pl.ds(hrows, hrows)],
                    gsems[buf],
                ),
            ]

        def out_copy(r, buf):
            return pltpu.make_async_copy(
                out_v.at[buf, :, pl.ds(0, dim0)],
                out_hbm.at[pl.ds((r * nw + wid) * ch, ch)],
                osems[buf],
            )

        def transpose_chunk(buf):
            def gr_body(gr, carry):
                d = (gr // hrows) * 8 + gr % 8
                jp = (gr % hrows) // 8
                cols = iota16 * 0 + d

                def s_body(s, rows):
                    val = in_v[buf, gr, pl.ds(s * dim0, dim0)]
                    plsc.store_scatter(out_v.at[buf], [rows, cols], val)
                    return rows + dim0

                lax.fori_loop(
                    0, LANES // dim0, s_body, jp * LANES + iota16, unroll=8
                )
                return carry

            lax.fori_loop(0, 2 * hrows, gr_body, 0)

        def transpose_tail(buf, n_cols):
            def col_loop(jp, nc):
                rows_j = rbase16 + jp * jw

                def cbody(c, lanes):
                    col = plsc.load_gather(in_v.at[buf], [rows_j, lanes])
                    out_v[
                        buf,
                        jp * LANES + c,
                        pl.ds(0, dim0),
                    ] = col
                    return lanes + 1

                lax.fori_loop(0, nc, cbody, iota16 * 0, unroll=8)

            for jp in range(-(-n_cols // LANES)):
                col_loop(jp, min(LANES, n_cols - jp * LANES))

        for cp in in_copies(0, 0):
            cp.start()

        for r in range(rounds):
            buf = r % 2
            nb = (r + 1) % 2

            @pl.when((r + 1) * nw + wid < n_chunk)
            def _():
                for cp in in_copies(r + 1, nb):
                    cp.start()

            if r >= 2:
                @pl.when((r - 2) * nw + wid < n_chunk)
                def _():
                    out_copy(r - 2, buf).wait()

            @pl.when(r * nw + wid < n_chunk)
            def _():
                for cp in in_copies(r, buf):
                    cp.wait()
                transpose_chunk(buf)
                out_copy(r, buf).start()

        for r in (rounds - 2, rounds - 1):

            @pl.when(r * nw + wid < n_chunk)
            def _():
                out_copy(r, r % 2).wait()

        @pl.when(wid == nw - 1)
        def _():
            pltpu.sync_copy(
                tlin_hbm.at[pl.ds(n_chunk * hrows, hrows)],
                in_v.at[0, pl.ds(0, hrows)],
            )
            pltpu.sync_copy(
                tlin_hbm.at[pl.ds(grows + n_chunk * hrows, hrows)],
                in_v.at[0, pl.ds(hrows, hrows)],
            )
            transpose_tail(0, vtail)
            pltpu.sync_copy(
                out_v.at[0, pl.ds(0, vtail), pl.ds(0, dim0)],
                out_hbm.at[pl.ds(vmain, vtail)],
            )

    return tkernel, jpad


def _build_gather(n_idx, dim0, num_cores, num_subcores):
    n_workers = num_cores * num_subcores
    per_w = n_idx // n_workers
    chunk = 2048
    while per_w % chunk != 0:
        chunk //= 2
    n_chunk = per_w // chunk

    mesh = plsc.VectorSubcoreMesh(core_axis_name="c", subcore_axis_name="s")

    @functools.partial(
        pl.kernel,
        mesh=mesh,
        compiler_params=pltpu.CompilerParams(use_tc_tiling_on_sc=False, needs_layout_passes=False),
        out_type=jax.ShapeDtypeStruct((n_idx, dim0), jnp.float32),
        scratch_types=[
            pltpu.VMEM((per_w,), jnp.int32),
            pltpu.VMEM((2, chunk, dim0), jnp.float32),
            pltpu.SemaphoreType.DMA,
            pltpu.SemaphoreType.DMA,
        ],
    )
    def gather(idx_hbm, table_hbm, emb_hbm, idx_v, rows_v, gsem0, gsem1):
        wid = lax.axis_index("s") * num_cores + lax.axis_index("c")
        base = wid * per_w
        pltpu.sync_copy(idx_hbm.at[pl.ds(base, per_w)], idx_v)
        gsems = [gsem0, gsem1]
        copies = [None, None]
        copies[0] = pltpu.async_copy(
            table_hbm.at[idx_v.at[pl.ds(0, chunk)]], rows_v.at[0], gsems[0]
        )
        for j in range(n_chunk):
            bj = j % 2
            if j + 1 < n_chunk:
                bn = (j + 1) % 2
                copies[bn] = pltpu.async_copy(
                    table_hbm.at[idx_v.at[pl.ds((j + 1) * chunk, chunk)]],
                    rows_v.at[bn],
                    gsems[bn],
                )
            copies[bj].wait()
            pltpu.sync_copy(
                rows_v.at[bj], emb_hbm.at[pl.ds(base + j * chunk, chunk)]
            )

    return gather


def _make_mm_body(dim0, dim1, pack, q_blk):
    def _mm_body(emb_ref, w_ref, b_ref, out_ref):
        for c2 in range(pack):
            sub = emb_ref[:, c2 * dim0 : (c2 + 1) * dim0]
            acc = lax.dot_general(
                w_ref[...],
                sub,
                (((0,), (1,)), ((), ())),
                preferred_element_type=jnp.float32,
            )
            out_ref[0, :, pl.ds(c2 * q_blk, q_blk)] = acc + b_ref[...]

    return _mm_body


def _upscale_t(emb128, w, b_col, seq, bsz, b_blk):
    dim0, dim1 = w.shape
    pack = 128 // dim0
    q_blk = b_blk // pack
    nb = bsz // b_blk
    return pl.pallas_call(
        _make_mm_body(dim0, dim1, pack, q_blk),
        grid=(seq, nb),
        in_specs=[
            pl.BlockSpec((q_blk, 128), lambda l, j: (l * nb + j, 0)),
            pl.BlockSpec((dim0, dim1), lambda l, j: (0, 0)),
            pl.BlockSpec((dim1, 1), lambda l, j: (0, 0)),
        ],
        out_specs=pl.BlockSpec((1, dim1, b_blk), lambda l, j: (l, 0, j)),
        out_shape=jax.ShapeDtypeStruct((seq, dim1, bsz), jnp.float32),
    )(emb128, w, b_col)


def kernel(x, table, W, b):
    bsz, seq = x.shape
    vocab, dim0 = table.shape
    dim1 = W.shape[1]
    n_idx = bsz * seq

    info = plsc.get_sparse_core_info()
    nc, ns = info.num_cores, info.num_subcores

    tkernel, jpad = _build_transpose(vocab, dim0, nc, ns)
    ttp = jnp.pad(table.T, ((0, 0), (0, jpad * LANES - vocab)))
    tlin = (
        ttp.reshape(2, dim0 // 2, jpad, LANES)
        .transpose(0, 2, 1, 3)
        .reshape(2 * jpad * dim0 // 2, LANES)
    )
    table_lin = tkernel(tlin).reshape(vocab, dim0)

    idx = x.T.reshape(-1).astype(jnp.int32)
    b_blk = 16384
    pack = 128 // dim0
    q_blk = b_blk // pack
    n_blocks = n_idx // b_blk
    idx_p = idx.reshape(n_blocks, pack, q_blk).swapaxes(1, 2).reshape(-1)
    gather = _build_gather(n_idx, dim0, nc, ns)
    emb = gather(idx_p, table_lin)
    emb128 = emb.reshape(n_idx // pack, pack * dim0)

    out3 = _upscale_t(emb128, W, b.reshape(dim1, 1), seq, bsz, b_blk=b_blk)
    return out3.transpose(2, 0, 1)


# speedup vs baseline: 1.2573x; 1.0014x over previous
import functools

import jax
import jax.numpy as jnp
from jax import lax
from jax.experimental import pallas as pl
from jax.experimental.pallas import tpu as pltpu
from jax.experimental.pallas import tpu_sc as plsc

LANES = 128


def _build_transpose(vocab, dim0, num_cores, num_subcores):
    nw = num_cores * num_subcores
    ch = 1024
    jw = ch // LANES
    vmain = (vocab // ch) * ch
    n_chunk = vmain // ch
    rounds = -(-n_chunk // nw)
    jpad = (n_chunk + 1) * jw
    vtail = vocab - vmain
    hrows = jw * (dim0 // 2)
    grows = jpad * (dim0 // 2)
    orows = ch * dim0 // LANES
    mesh = plsc.VectorSubcoreMesh(core_axis_name="c", subcore_axis_name="s")

    @functools.partial(
        pl.kernel,
        mesh=mesh,
        compiler_params=pltpu.CompilerParams(
            use_tc_tiling_on_sc=False, needs_layout_passes=False
        ),
        out_type=jax.ShapeDtypeStruct((vocab, dim0), jnp.float32),
        scratch_types=[
            pltpu.VMEM((2, 2 * hrows, LANES), jnp.float32),
            pltpu.VMEM((2, ch, dim0 + 1), jnp.float32),
            pltpu.SemaphoreType.DMA,
            pltpu.SemaphoreType.DMA,
            pltpu.SemaphoreType.DMA,
            pltpu.SemaphoreType.DMA,
        ],
    )
    def tkernel(tlin_hbm, out_hbm, in_v, out_v, gs0, gs1, os0, os1):
        wid = lax.axis_index("s") * num_cores + lax.axis_index("c")
        iota16 = lax.iota(jnp.int32, dim0)
        rbase16 = (iota16 // 8) * hrows + (iota16 % 8)
        gsems = [gs0, gs1]
        osems = [os0, os1]

        def in_copies(r, buf):
            off = (r * nw + wid) * hrows
            return [
                pltpu.make_async_copy(
                    tlin_hbm.at[pl.ds(off, hrows)],
                    in_v.at[buf, pl.ds(0, hrows)],
                    gsems[buf],
                ),
                pltpu.make_async_copy(
                    tlin_hbm.at[pl.ds(grows + off, hrows)],
                    in_v.at[buf, pl.ds(hrows, hrows)],
                    gsems[buf],
                ),
            ]

        def out_copy(r, buf):
            return pltpu.make_async_copy(
                out_v.at[buf, :, pl.ds(0, dim0)],
                out_hbm.at[pl.ds((r * nw + wid) * ch, ch)],
                osems[buf],
            )

        def transpose_chunk(buf):
            def gr_body(gr, carry):
                d = (gr // hrows) * 8 + gr % 8
                jp = (gr % hrows) // 8
                cols = iota16 * 0 + d
                rows = jp * LANES + iota16
                for s in range(LANES // dim0):
                    val = in_v[buf, gr, pl.ds(s * dim0, dim0)]
                    plsc.store_scatter(out_v.at[buf], [rows, cols], val)
                    if s + 1 < LANES // dim0:
                        rows = rows + dim0
                return carry

            lax.fori_loop(0, 2 * hrows, gr_body, 0)

        def transpose_tail(buf, n_cols):
            def col_loop(jp, nc):
                rows_j = rbase16 + jp * jw

                def cbody(c, lanes):
                    col = plsc.load_gather(in_v.at[buf], [rows_j, lanes])
                    out_v[
                        buf,
                        jp * LANES + c,
                        pl.ds(0, dim0),
                    ] = col
                    return lanes + 1

                lax.fori_loop(0, nc, cbody, iota16 * 0, unroll=8)

            for jp in range(-(-n_cols // LANES)):
                col_loop(jp, min(LANES, n_cols - jp * LANES))

        for cp in in_copies(0, 0):
            cp.start()

        for r in range(rounds):
            buf = r % 2
            nb = (r + 1) % 2

            @pl.when((r + 1) * nw + wid < n_chunk)
            def _():
                for cp in in_copies(r + 1, nb):
                    cp.start()

            if r >= 2:
                @pl.when((r - 2) * nw + wid < n_chunk)
                def _():
                    out_copy(r - 2, buf).wait()

            @pl.when(r * nw + wid < n_chunk)
            def _():
                for cp in in_copies(r, buf):
                    cp.wait()
                transpose_chunk(buf)
                out_copy(r, buf).start()

        for r in (rounds - 2, rounds - 1):

            @pl.when(r * nw + wid < n_chunk)
            def _():
                out_copy(r, r % 2).wait()

        @pl.when(wid == nw - 1)
        def _():
            pltpu.sync_copy(
                tlin_hbm.at[pl.ds(n_chunk * hrows, hrows)],
                in_v.at[0, pl.ds(0, hrows)],
            )
            pltpu.sync_copy(
                tlin_hbm.at[pl.ds(grows + n_chunk * hrows, hrows)],
                in_v.at[0, pl.ds(hrows, hrows)],
            )
            transpose_tail(0, vtail)
            pltpu.sync_copy(
                out_v.at[0, pl.ds(0, vtail), pl.ds(0, dim0)],
                out_hbm.at[pl.ds(vmain, vtail)],
            )

    return tkernel, jpad


def _build_gather(n_idx, dim0, num_cores, num_subcores):
    n_workers = num_cores * num_subcores
    per_w = n_idx // n_workers
    chunk = 2048
    while per_w % chunk != 0:
        chunk //= 2
    n_chunk = per_w // chunk

    mesh = plsc.VectorSubcoreMesh(core_axis_name="c", subcore_axis_name="s")

    @functools.partial(
        pl.kernel,
        mesh=mesh,
        compiler_params=pltpu.CompilerParams(use_tc_tiling_on_sc=False, needs_layout_passes=False),
        out_type=jax.ShapeDtypeStruct((n_idx, dim0), jnp.float32),
        scratch_types=[
            pltpu.VMEM((per_w,), jnp.int32),
            pltpu.VMEM((2, chunk, dim0), jnp.float32),
            pltpu.SemaphoreType.DMA,
            pltpu.SemaphoreType.DMA,
        ],
    )
    def gather(idx_hbm, table_hbm, emb_hbm, idx_v, rows_v, gsem0, gsem1):
        wid = lax.axis_index("s") * num_cores + lax.axis_index("c")
        base = wid * per_w
        pltpu.sync_copy(idx_hbm.at[pl.ds(base, per_w)], idx_v)
        gsems = [gsem0, gsem1]
        copies = [None, None]
        copies[0] = pltpu.async_copy(
            table_hbm.at[idx_v.at[pl.ds(0, chunk)]], rows_v.at[0], gsems[0]
        )
        for j in range(n_chunk):
            bj = j % 2
            if j + 1 < n_chunk:
                bn = (j + 1) % 2
                copies[bn] = pltpu.async_copy(
                    table_hbm.at[idx_v.at[pl.ds((j + 1) * chunk, chunk)]],
                    rows_v.at[bn],
                    gsems[bn],
                )
            copies[bj].wait()
            pltpu.sync_copy(
                rows_v.at[bj], emb_hbm.at[pl.ds(base + j * chunk, chunk)]
            )

    return gather


def _make_mm_body(dim0, dim1, pack, q_blk):
    def _mm_body(emb_ref, w_ref, b_ref, out_ref):
        for c2 in range(pack):
            sub = emb_ref[:, c2 * dim0 : (c2 + 1) * dim0]
            acc = lax.dot_general(
                w_ref[...],
                sub,
                (((0,), (1,)), ((), ())),
                preferred_element_type=jnp.float32,
            )
            out_ref[0, :, pl.ds(c2 * q_blk, q_blk)] = acc + b_ref[...]

    return _mm_body


def _upscale_t(emb128, w, b_col, seq, bsz, b_blk):
    dim0, dim1 = w.shape
    pack = 128 // dim0
    q_blk = b_blk // pack
    nb = bsz // b_blk
    return pl.pallas_call(
        _make_mm_body(dim0, dim1, pack, q_blk),
        grid=(seq, nb),
        in_specs=[
            pl.BlockSpec((q_blk, 128), lambda l, j: (l * nb + j, 0)),
            pl.BlockSpec((dim0, dim1), lambda l, j: (0, 0)),
            pl.BlockSpec((dim1, 1), lambda l, j: (0, 0)),
        ],
        out_specs=pl.BlockSpec((1, dim1, b_blk), lambda l, j: (l, 0, j)),
        out_shape=jax.ShapeDtypeStruct((seq, dim1, bsz), jnp.float32),
    )(emb128, w, b_col)


def kernel(x, table, W, b):
    bsz, seq = x.shape
    vocab, dim0 = table.shape
    dim1 = W.shape[1]
    n_idx = bsz * seq

    info = plsc.get_sparse_core_info()
    nc, ns = info.num_cores, info.num_subcores

    tkernel, jpad = _build_transpose(vocab, dim0, nc, ns)
    ttp = jnp.pad(table.T, ((0, 0), (0, jpad * LANES - vocab)))
    tlin = (
        ttp.reshape(2, dim0 // 2, jpad, LANES)
        .transpose(0, 2, 1, 3)
        .reshape(2 * jpad * dim0 // 2, LANES)
    )
    table_lin = tkernel(tlin).reshape(vocab, dim0)

    idx = x.T.reshape(-1).astype(jnp.int32)
    b_blk = 16384
    pack = 128 // dim0
    q_blk = b_blk // pack
    n_blocks = n_idx // b_blk
    idx_p = idx.reshape(n_blocks, pack, q_blk).swapaxes(1, 2).reshape(-1)
    gather = _build_gather(n_idx, dim0, nc, ns)
    emb = gather(idx_p, table_lin)
    emb128 = emb.reshape(n_idx // pack, pack * dim0)

    out3 = _upscale_t(emb128, W, b.reshape(dim1, 1), seq, bsz, b_blk=b_blk)
    return out3.transpose(2, 0, 1)
